# Initial kernel scaffold; baseline (speedup 1.0000x reference)
import jax
import jax.numpy as jnp
from jax.experimental import pallas as pl


def kernel(text, label, embed_table, gamma1, beta1, W1, b1, gamma2, beta2, W2, b2):
    raise NotImplementedError("write your pallas kernel here")


# speedup vs baseline: 9.3658x; 9.3658x over previous
import functools

import jax
import jax.numpy as jnp
from jax import lax
from jax.experimental import pallas as pl
from jax.experimental.pallas import tpu as pltpu
from jax.experimental.pallas import tpu_sc as plsc

VOCAB_ = 100000
EMBED_ = 128
HIDDEN_ = 1024
OUT_ = 2
SEQ_ = 20
BATCH_ = 16384
EPS_ = 1e-5

_NC = 2
_NS = 16
_NW = _NC * _NS
_BPW = BATCH_ // _NW
_CH = 128
_NCH = _BPW // _CH


def _pool_body(text_hbm, table_hbm, out_hbm, idx_v, acc_v, sems):
    wid = lax.axis_index("s") * _NC + lax.axis_index("c")
    base = wid * _BPW
    for c in range(_NCH):
        pltpu.sync_copy(text_hbm.at[:, pl.ds(base + c * _CH, _CH)],
                        idx_v.at[c])
    for s in range(SEQ_):
        cps = [pltpu.async_copy(table_hbm.at[idx_v.at[c].at[s]],
                                acc_v.at[c], sems.at[c], add=(s > 0))
               for c in range(_NCH)]
        for cp in cps:
            cp.wait()
    for c in range(_NCH):
        pltpu.sync_copy(acc_v.at[c], out_hbm.at[pl.ds(base + c * _CH, _CH)])


def _pool(text, table):
    mesh = plsc.VectorSubcoreMesh(core_axis_name="c", subcore_axis_name="s")
    return pl.kernel(
        _pool_body,
        out_type=jax.ShapeDtypeStruct((BATCH_, EMBED_), jnp.float32),
        mesh=mesh,
        scratch_types=[
            pltpu.VMEM((_NCH, SEQ_, _CH), jnp.int32),
            pltpu.VMEM((_NCH, _CH, EMBED_), jnp.float32),
            pltpu.SemaphoreType.DMA((_NCH,)),
        ],
    )(text, table)


_BB = 2048


def _stats_body(x_ref, g_ref, s_ref):
    i = pl.program_id(0)
    x = x_ref[...]
    xtx = lax.dot_general(x, x, (((0,), (0,)), ((), ())),
                          preferred_element_type=jnp.float32)
    cs = jnp.sum(x, axis=0, keepdims=True)

    @pl.when(i == 0)
    def _():
        g_ref[...] = xtx
        s_ref[...] = cs

    @pl.when(i > 0)
    def _():
        g_ref[...] += xtx
        s_ref[...] += cs


def _stats(psum):
    return pl.pallas_call(
        _stats_body,
        grid=(BATCH_ // _BB,),
        in_specs=[pl.BlockSpec((_BB, EMBED_), lambda i: (i, 0))],
        out_specs=[
            pl.BlockSpec((EMBED_, EMBED_), lambda i: (0, 0)),
            pl.BlockSpec((1, EMBED_), lambda i: (0, 0)),
        ],
        out_shape=[
            jax.ShapeDtypeStruct((EMBED_, EMBED_), jnp.float32),
            jax.ShapeDtypeStruct((1, EMBED_), jnp.float32),
        ],
    )(psum)


def _eye(n):
    r = lax.broadcasted_iota(jnp.int32, (n, n), 0)
    c = lax.broadcasted_iota(jnp.int32, (n, n), 1)
    return (r == c).astype(jnp.float32)


def _fold_body(g_ref, s_ref, g1_ref, be1_ref, w1_ref, b1_ref,
               g2_ref, be2_ref, w2_ref, b2_ref, m_ref, d_ref):
    G = g_ref[...]
    ssum = s_ref[...]
    g1 = g1_ref[...]
    be1 = be1_ref[...]
    W1 = w1_ref[...]
    b1 = b1_ref[...]
    g2 = g2_ref[...]
    be2 = be2_ref[...]
    W2 = w2_ref[...]
    b2 = b2_ref[...]

    dot = functools.partial(lax.dot_general,
                            preferred_element_type=jnp.float32)
    mm = lambda a, b: dot(a, b, (((1,), (0,)), ((), ())))
    outer = lambda a, b: dot(a, b, (((0,), (0,)), ((), ())))

    mu = ssum * (1.0 / (SEQ_ * BATCH_))
    Cov = G * (1.0 / (SEQ_ * SEQ_ * BATCH_)) - outer(mu, mu)
    e128 = _eye(EMBED_)
    var1 = jnp.sum(Cov * e128, axis=0, keepdims=True)
    a1 = g1 * lax.rsqrt(var1 + EPS_)
    c1 = be1 - mu * a1
    CovA = Cov * outer(a1, a1)
    T = mm(CovA, W1)
    varh = jnp.sum(W1 * T, axis=0, keepdims=True)
    muh = mm(be1, W1) + b1
    a2 = g2 * lax.rsqrt(varh + EPS_)
    c2 = be2 - muh * a2
    W2e = mm(_eye(HIDDEN_) * a2, W2)
    K = mm(W1, W2e)
    M = mm(e128 * a1, K)
    b1e = mm(c1, W1) + b1
    d = mm(b1e, W2e) + mm(c2, W2) + b2
    m_ref[...] = M * (1.0 / SEQ_)
    d_ref[...] = d


def _fold(G, ssum, g1, be1, W1, b1, g2, be2, W2, b2):
    return pl.pallas_call(
        _fold_body,
        out_shape=[
            jax.ShapeDtypeStruct((EMBED_, OUT_), jnp.float32),
            jax.ShapeDtypeStruct((1, OUT_), jnp.float32),
        ],
    )(G, ssum, g1, be1, W1, b1, g2, be2, W2, b2)


def _proj_body(x_ref, m_ref, d_ref, o_ref):
    o_ref[...] = lax.dot_general(
        x_ref[...], m_ref[...], (((1,), (0,)), ((), ())),
        preferred_element_type=jnp.float32) + d_ref[...]


def _proj(psum, M, d):
    return pl.pallas_call(
        _proj_body,
        grid=(BATCH_ // _BB,),
        in_specs=[
            pl.BlockSpec((_BB, EMBED_), lambda i: (i, 0)),
            pl.BlockSpec((EMBED_, OUT_), lambda i: (0, 0)),
            pl.BlockSpec((1, OUT_), lambda i: (0, 0)),
        ],
        out_specs=pl.BlockSpec((_BB, OUT_), lambda i: (i, 0)),
        out_shape=jax.ShapeDtypeStruct((BATCH_, OUT_), jnp.float32),
    )(psum, M, d)


def kernel(text, label, embed_table, gamma1, beta1, W1, b1,
           gamma2, beta2, W2, b2):
    del label
    psum = _pool(text, embed_table)
    G, ssum = _stats(psum)
    M, d = _fold(G, ssum,
                 gamma1.reshape(1, -1), beta1.reshape(1, -1), W1,
                 b1.reshape(1, -1), gamma2.reshape(1, -1),
                 beta2.reshape(1, -1), W2, b2.reshape(1, -1))
    return _proj(psum, M, d)
